# baseline (speedup 1.0000x reference)
import functools

import jax
import jax.numpy as jnp
from jax import lax
from jax.experimental import pallas as pl
from jax.experimental.pallas import tpu as pltpu
from jax.experimental.pallas import tpu_sc as plsc

B, N = 64, 32768
D, K = 8192, 8
NC, NS = 2, 16
BH = B // NC
NPT = N // NS
NB = 256
G = 64
GR = G * K
NJ = GR // 128
NG = D // NS // G


def _body(x, det, out, x2, stat, *rest):
    idx_flat, rest = rest[:3 * 2 * NJ], rest[3 * 2 * NJ:]
    idxr = [list(idx_flat[0:NJ]), list(idx_flat[NJ:2 * NJ])]
    idxg = [list(idx_flat[2 * NJ:3 * NJ]), list(idx_flat[3 * NJ:4 * NJ])]
    idxs = [list(idx_flat[4 * NJ:5 * NJ]), list(idx_flat[5 * NJ:6 * NJ])]
    vals0, vals1, flags0, flags1, xb, tb, zbuf = rest[:7]
    semg0, semg1, sems0, sems1 = rest[7:]
    vals = [vals0, vals1]
    flags = [flags0, flags1]
    semg = [semg0, semg1]
    sems = [sems0, sems1]

    c = lax.axis_index("c")
    s = lax.axis_index("s")
    cn = c * N
    n0 = s * NPT
    iota = lax.iota(jnp.int32, 16)
    zero16 = jnp.zeros((16,), jnp.int32)

    def zrow(r, cc):
        zbuf[r, :] = jnp.zeros((BH,), jnp.int16)
        return cc

    lax.fori_loop(0, NB, zrow, 0)

    def zblk(j, cc):
        pltpu.sync_copy(zbuf, stat.at[pl.ds(n0 + j * NB, NB)])
        return cc

    lax.fori_loop(0, NPT // NB, zblk, 0)

    def xblk(j, cc):
        nb = n0 + j * NB
        pltpu.sync_copy(x.at[pl.ds(c * BH, BH), pl.ds(nb, NB)], xb)

        def trow(b, cc2):
            col = zero16 + b
            for t in range(NB // 16):
                plsc.store_scatter(tb, [iota + t * 16, col],
                                   xb[b, pl.ds(t * 16, 16)])
            return cc2

        lax.fori_loop(0, BH, trow, 0)
        pltpu.sync_copy(tb, x2.at[pl.ds(cn + nb, NB)])
        return cc

    lax.fori_loop(0, NPT // NB, xblk, 0)
    plsc.subcore_barrier()

    def load_idx(p, g):
        off = (s * (D // NS) + g * G) * K
        for j in range(NJ):
            pltpu.sync_copy(det.at[pl.ds(off + j * 128, 128)], idxr[p][j])
        for j in range(NJ):
            for t in range(8):
                sl = pl.ds(t * 16, 16)
                idxg[p][j][sl] = idxr[p][j][sl] + cn

    def issue_gather(p):
        for j in range(NJ):
            pltpu.async_copy(x2.at[idxg[p][j]],
                             vals[p].at[pl.ds(j * 128, 128)], semg[p])

    def wait_gather(p):
        for j in range(NJ):
            pltpu.make_async_copy(x2.at[idxg[p][j]],
                                  vals[p].at[pl.ds(j * 128, 128)],
                                  semg[p]).wait()

    def issue_scatter(p):
        for j in range(NJ):
            for t in range(8):
                sl = pl.ds(t * 16, 16)
                idxs[p][j][sl] = idxr[p][j][sl]
        for j in range(NJ):
            pltpu.async_copy(flags[p].at[pl.ds(j * 128, 128)],
                             stat.at[idxs[p][j]], sems[p], add=True)

    def wait_scatter(p):
        for j in range(NJ):
            pltpu.make_async_copy(flags[p].at[pl.ds(j * 128, 128)],
                                  stat.at[idxs[p][j]], sems[p]).wait()

    def compute(p):
        vp = vals[p]
        fp = flags[p]

        def det_body(d, cc):
            r0 = d * K
            packed = []
            for h in (0, 1):
                sl = pl.ds(h * 16, 16)
                v = [vp[r0 + k, sl] for k in range(K)]
                m = v[0]
                for k in range(1, K):
                    m = jnp.maximum(m, v[k])
                eq = v[0] == m
                wins = [eq]
                seen = eq
                for k in range(1, K):
                    eq = v[k] == m
                    wins.append(eq & ~seen)
                    seen = seen | eq
                packed.append([jnp.where(w, 0, 1).astype(jnp.int32)
                               for w in wins])
            for k in range(K):
                both = packed[0][k] | lax.shift_left(packed[1][k], 16)
                fp[r0 + k, :] = plsc.bitcast(both, jnp.int16)
            return cc

        lax.fori_loop(0, G, det_body, 0)

    load_idx(0, 0)
    issue_gather(0)

    def pair(i, cc):
        ga = 2 * i
        load_idx(1, ga + 1)
        issue_gather(1)
        wait_gather(0)

        @pl.when(i > 0)
        def _():
            wait_scatter(0)

        compute(0)
        issue_scatter(0)

        @pl.when(i < NG // 2 - 1)
        def _():
            load_idx(0, ga + 2)
            issue_gather(0)

        wait_gather(1)

        @pl.when(i > 0)
        def _():
            wait_scatter(1)

        compute(1)
        issue_scatter(1)
        return cc

    lax.fori_loop(0, NG // 2, pair, 0)
    wait_scatter(0)
    wait_scatter(1)
    plsc.subcore_barrier()

    def oblk(j, cc):
        nb = n0 + j * NB
        pltpu.sync_copy(stat.at[pl.ds(nb, NB)], flags0.at[pl.ds(0, NB)])
        pltpu.sync_copy(x2.at[pl.ds(cn + nb, NB)], vals0.at[pl.ds(0, NB)])

        def orow(n, cc2):
            w = plsc.bitcast(flags0[n, :], jnp.int32)
            a = w & 0xFFFF
            b = lax.shift_right_logical(w, 16)
            o0 = jnp.where(a == 0, vals0[n, pl.ds(0, 16)], 0.0)
            o1 = jnp.where(b == 0, vals0[n, pl.ds(16, 16)], 0.0)
            col = zero16 + n
            plsc.store_scatter(xb, [iota, col], o0)
            plsc.store_scatter(xb, [iota + 16, col], o1)
            return cc2

        lax.fori_loop(0, NB, orow, 0)
        pltpu.sync_copy(xb, out.at[pl.ds(c * BH, BH), pl.ds(nb, NB)])
        return cc

    lax.fori_loop(0, NPT // NB, oblk, 0)


_sc_call = functools.partial(
    pl.kernel,
    out_type=jax.ShapeDtypeStruct((B, N), jnp.float32),
    mesh=plsc.VectorSubcoreMesh(core_axis_name="c", subcore_axis_name="s"),
    compiler_params=pltpu.CompilerParams(
        needs_layout_passes=False, use_tc_tiling_on_sc=False),
    scratch_types=(
        [pltpu.HBM((NC * N, BH), jnp.float32),
         pltpu.VMEM_SHARED((N, BH), jnp.int16)]
        + [pltpu.VMEM((128,), jnp.int32)] * (3 * 2 * NJ)
        + [pltpu.VMEM((GR, BH), jnp.float32),
           pltpu.VMEM((GR, BH), jnp.float32),
           pltpu.VMEM((GR, BH), jnp.int16),
           pltpu.VMEM((GR, BH), jnp.int16),
           pltpu.VMEM((BH, NB), jnp.float32),
           pltpu.VMEM((NB, BH), jnp.float32),
           pltpu.VMEM((NB, BH), jnp.int16),
           pltpu.SemaphoreType.DMA,
           pltpu.SemaphoreType.DMA,
           pltpu.SemaphoreType.DMA,
           pltpu.SemaphoreType.DMA]
    ),
)(_body)


@jax.jit
def kernel(x, detectors):
    return _sc_call(x, detectors.reshape(-1))


# speedup vs baseline: 1.2618x; 1.2618x over previous
import functools

import jax
import jax.numpy as jnp
from jax import lax
from jax.experimental import pallas as pl
from jax.experimental.pallas import tpu as pltpu
from jax.experimental.pallas import tpu_sc as plsc

B, N = 64, 32768
D, K = 8192, 8
NC, NS = 2, 16
BH = B // NC
DPT = D // NS
GD = 64
GR = GD * K
NJ = GR // 128
NG = DPT // GD
CH = 16
RPT = N // NS
RB = 128


def _body(x2, det, out, stat, *rest):
    idx_flat, rest = rest[:3 * 2 * NJ], rest[3 * 2 * NJ:]
    idxr = [list(idx_flat[0:NJ]), list(idx_flat[NJ:2 * NJ])]
    idxg = [list(idx_flat[2 * NJ:3 * NJ]), list(idx_flat[3 * NJ:4 * NJ])]
    idxs = [list(idx_flat[4 * NJ:5 * NJ]), list(idx_flat[5 * NJ:6 * NJ])]
    vals0, vals1, flags0, flags1, zbuf, semg0, semg1, sems0, sems1 = rest
    vals = [vals0, vals1]
    flags = [flags0, flags1]
    semg = [semg0, semg1]
    sems = [sems0, sems1]

    c = lax.axis_index("c")
    s = lax.axis_index("s")
    cn = c * N

    def load_idx(p, g):
        off = s * (DPT * K) + g * GR
        for j in range(NJ):
            pltpu.sync_copy(det.at[pl.ds(off + j * 128, 128)], idxr[p][j])
        for j in range(NJ):
            for t in range(8):
                sl = pl.ds(t * 16, 16)
                idxg[p][j][sl] = idxr[p][j][sl] + cn

    def issue_gather(p):
        for j in range(NJ):
            pltpu.async_copy(x2.at[idxg[p][j]],
                             vals[p].at[pl.ds(j * 128, 128)], semg[p])

    def wait_gather(p):
        for j in range(NJ):
            pltpu.make_async_copy(x2.at[idxg[p][j]],
                                  vals[p].at[pl.ds(j * 128, 128)],
                                  semg[p]).wait()

    def issue_scatter(p):
        for j in range(NJ):
            for t in range(8):
                sl = pl.ds(t * 16, 16)
                idxs[p][j][sl] = idxr[p][j][sl]
        for j in range(NJ):
            pltpu.async_copy(flags[p].at[pl.ds(j * 128, 128)],
                             stat.at[idxs[p][j]], sems[p], add=True)

    def wait_scatter(p):
        for j in range(NJ):
            pltpu.make_async_copy(flags[p].at[pl.ds(j * 128, 128)],
                                  stat.at[idxs[p][j]], sems[p]).wait()

    def compute(p):
        vp = vals[p]
        fp = flags[p]

        def chunk_body(q, cc):
            cb = q * (CH * K)
            for g in range(CH):
                r0 = g * K
                packed = []
                for h in (0, 1):
                    sl = pl.ds(h * 16, 16)
                    v = [vp[cb + r0 + k, sl] for k in range(K)]
                    m = v[0]
                    for k in range(1, K):
                        m = jnp.maximum(m, v[k])
                    eq = v[0] == m
                    wins = [eq]
                    seen = eq
                    for k in range(1, K):
                        eq = v[k] == m
                        wins.append(eq & ~seen)
                        seen = seen | eq
                    packed.append([jnp.where(w, 0, 1).astype(jnp.int32)
                                   for w in wins])
                for k in range(K):
                    both = packed[0][k] | lax.shift_left(packed[1][k], 16)
                    fp[cb + r0 + k, :] = plsc.bitcast(both, jnp.int16)
            return cc

        lax.fori_loop(0, GD // CH, chunk_body, 0)

    load_idx(0, 0)
    issue_gather(0)

    for i in range(RB):
        zbuf[i, :] = jnp.zeros((BH,), jnp.int16)

    def zero_blk(j, cc):
        pltpu.sync_copy(zbuf, stat.at[pl.ds(s * RPT + j * RB, RB)])
        return cc

    lax.fori_loop(0, RPT // RB, zero_blk, 0)
    plsc.subcore_barrier()

    def pair(i, cc):
        ga = 2 * i
        load_idx(1, ga + 1)
        issue_gather(1)
        wait_gather(0)

        @pl.when(i > 0)
        def _():
            wait_scatter(0)

        compute(0)
        issue_scatter(0)

        @pl.when(i < NG // 2 - 1)
        def _():
            load_idx(0, ga + 2)
            issue_gather(0)

        wait_gather(1)

        @pl.when(i > 0)
        def _():
            wait_scatter(1)

        compute(1)
        issue_scatter(1)
        return cc

    lax.fori_loop(0, NG // 2, pair, 0)
    wait_scatter(0)
    wait_scatter(1)
    plsc.subcore_barrier()

    def out_blk(i, cc):
        r0 = s * RPT + i * RB
        pltpu.sync_copy(stat.at[pl.ds(r0, RB)], flags0.at[pl.ds(0, RB)])
        pltpu.sync_copy(x2.at[pl.ds(cn + r0, RB)], vals0.at[pl.ds(0, RB)])
        for r in range(RB):
            w = plsc.bitcast(flags0[r, :], jnp.int32)
            a = w & 0xFFFF
            b = lax.shift_right_logical(w, 16)
            x0 = vals0[r, pl.ds(0, 16)]
            x1 = vals0[r, pl.ds(16, 16)]
            vals0[r, pl.ds(0, 16)] = jnp.where(a == 0, x0, 0.0)
            vals0[r, pl.ds(16, 16)] = jnp.where(b == 0, x1, 0.0)
        pltpu.sync_copy(vals0.at[pl.ds(0, RB)], out.at[pl.ds(cn + r0, RB)])
        return cc

    lax.fori_loop(0, RPT // RB, out_blk, 0)


_sc_call = functools.partial(
    pl.kernel,
    out_type=jax.ShapeDtypeStruct((NC * N, BH), jnp.float32),
    mesh=plsc.VectorSubcoreMesh(core_axis_name="c", subcore_axis_name="s"),
    compiler_params=pltpu.CompilerParams(
        needs_layout_passes=False, use_tc_tiling_on_sc=False),
    scratch_types=(
        [pltpu.VMEM_SHARED((N, BH), jnp.int16)]
        + [pltpu.VMEM((128,), jnp.int32)] * (3 * 2 * NJ)
        + [pltpu.VMEM((GR, BH), jnp.float32),
           pltpu.VMEM((GR, BH), jnp.float32),
           pltpu.VMEM((GR, BH), jnp.int16),
           pltpu.VMEM((GR, BH), jnp.int16),
           pltpu.VMEM((RB, BH), jnp.int16),
           pltpu.SemaphoreType.DMA,
           pltpu.SemaphoreType.DMA,
           pltpu.SemaphoreType.DMA,
           pltpu.SemaphoreType.DMA]
    ),
)(_body)


@jax.jit
def kernel(x, detectors):
    x2 = x.reshape(NC, BH, N).transpose(0, 2, 1).reshape(NC * N, BH)
    det = detectors.reshape(-1)
    out2 = _sc_call(x2, det)
    return out2.reshape(NC, N, BH).transpose(0, 2, 1).reshape(B, N)
